# Initial kernel scaffold; baseline (speedup 1.0000x reference)
import jax
import jax.numpy as jnp
from jax.experimental import pallas as pl


def kernel(x, edge_index, W_enc, b_enc, ln_g, ln_b, t, W1, b1, mg, mb, W2, b2, W_out, b_out):
    raise NotImplementedError("write your pallas kernel here")


# speedup vs baseline: 5.2627x; 5.2627x over previous
import functools

import jax
import jax.numpy as jnp
from jax import lax
from jax.experimental import pallas as pl
from jax.experimental.pallas import tpu as pltpu
from jax.experimental.pallas import tpu_sc as plsc

N = 10000
E = 320000
H = 128
DIN = 1070
DPAD = 1152

NSUB = 16
CHUNK = 128
NCHUNK = 160
GSTAGE = 32
NSTAGE = NCHUNK // GSTAGE
EPAD = NSUB * NCHUNK * CHUNK
ROWS_PER_SUB = 632
ACC_ROWS = ROWS_PER_SUB * NSUB

ROW_BLK = 1000
NBLK = N // ROW_BLK


def _enc_body(x_ref, w_ref, b_ref, o_ref):
    o_ref[...] = (
        jnp.dot(x_ref[...], w_ref[...], preferred_element_type=jnp.float32, precision=lax.Precision.HIGHEST)
        + b_ref[...]
    )


def _encoder(xp, wp, b):
    return pl.pallas_call(
        _enc_body,
        grid=(NBLK,),
        in_specs=[
            pl.BlockSpec((ROW_BLK, DPAD), lambda i: (i, 0)),
            pl.BlockSpec((DPAD, H), lambda i: (0, 0)),
            pl.BlockSpec((1, H), lambda i: (0, 0)),
        ],
        out_specs=pl.BlockSpec((ROW_BLK, H), lambda i: (i, 0)),
        out_shape=jax.ShapeDtypeStruct((N, H), jnp.float32),
    )(xp, wp, b)


def _ln(v, g, b):
    mu = jnp.mean(v, axis=-1, keepdims=True)
    var = jnp.mean((v - mu) ** 2, axis=-1, keepdims=True)
    return (v - mu) * lax.rsqrt(var + 1e-5) * g + b


def _prep1_body(h_ref, g_ref, b_ref, t_ref, z_ref, gb_ref, *, first):
    h = h_ref[...]
    if first:
        z = h
    else:
        z = jnp.maximum(_ln(h, g_ref[...], b_ref[...]), 0.0)
    z_ref[...] = z
    m = jnp.maximum(z, 0.0) + 1e-7
    gb_ref[...] = jnp.max(m * t_ref[...], axis=0, keepdims=True).reshape(1, 1, -1)


def _prep1(h, g, b, tb, first):
    return pl.pallas_call(
        functools.partial(_prep1_body, first=first),
        grid=(NBLK,),
        in_specs=[
            pl.BlockSpec((ROW_BLK, H), lambda i: (i, 0)),
            pl.BlockSpec((1, H), lambda i: (0, 0)),
            pl.BlockSpec((1, H), lambda i: (0, 0)),
            pl.BlockSpec((1, H), lambda i: (0, 0)),
        ],
        out_specs=[
            pl.BlockSpec((ROW_BLK, H), lambda i: (i, 0)),
            pl.BlockSpec((1, 1, H), lambda i: (i, 0, 0)),
        ],
        out_shape=[
            jax.ShapeDtypeStruct((N, H), jnp.float32),
            jax.ShapeDtypeStruct((NBLK, 1, H), jnp.float32),
        ],
    )(h, g, b, tb)


def _prep2_body(z_ref, gb_ref, t_ref, T_ref):
    z = z_ref[...]
    m = jnp.maximum(z, 0.0) + 1e-7
    logit = m * t_ref[...]
    G = jnp.max(gb_ref[...].reshape(-1, gb_ref.shape[-1]), axis=0, keepdims=True)
    P = jnp.exp(logit - G)
    Q = m * P
    T_ref[0, :, :64] = P[:, :64]
    T_ref[0, :, 64:] = Q[:, :64]
    T_ref[1, :, :64] = P[:, 64:]
    T_ref[1, :, 64:] = Q[:, 64:]


def _prep2(z, gb, tb):
    return pl.pallas_call(
        _prep2_body,
        grid=(NBLK,),
        in_specs=[
            pl.BlockSpec((ROW_BLK, H), lambda i: (i, 0)),
            pl.BlockSpec((NBLK, 1, H), lambda i: (0, 0, 0)),
            pl.BlockSpec((1, H), lambda i: (0, 0)),
        ],
        out_specs=pl.BlockSpec((2, ROW_BLK, H), lambda i: (0, i, 0)),
        out_shape=jax.ShapeDtypeStruct((2, N, H), jnp.float32),
    )(z, gb, tb)


def _post_body(o0_ref, o1_ref, z_ref, h_ref, w1_ref, b1_ref, mg_ref, mb_ref,
               w2_ref, b2_ref, out_ref, *, residual):
    den = jnp.concatenate([o0_ref[0, :, :64], o1_ref[0, :, :64]], axis=1)
    num = jnp.concatenate([o0_ref[0, :, 64:], o1_ref[0, :, 64:]], axis=1)
    aggr = num / (den + 1e-16)
    hm = aggr + z_ref[...]
    u = jnp.dot(hm, w1_ref[...], preferred_element_type=jnp.float32, precision=lax.Precision.HIGHEST) + b1_ref[...]
    u = jnp.maximum(_ln(u, mg_ref[...], mb_ref[...]), 0.0)
    y = jnp.dot(u, w2_ref[...], preferred_element_type=jnp.float32, precision=lax.Precision.HIGHEST) + b2_ref[...]
    if residual:
        y = y + h_ref[...]
    out_ref[...] = y


def _post(O, z, h, w1, b1, mg, mb, w2, b2, residual):
    return pl.pallas_call(
        functools.partial(_post_body, residual=residual),
        grid=(NBLK,),
        in_specs=[
            pl.BlockSpec((1, ROW_BLK, H), lambda i: (0, i, 0)),
            pl.BlockSpec((1, ROW_BLK, H), lambda i: (1, i, 0)),
            pl.BlockSpec((ROW_BLK, H), lambda i: (i, 0)),
            pl.BlockSpec((ROW_BLK, H), lambda i: (i, 0)),
            pl.BlockSpec((H, 2 * H), lambda i: (0, 0)),
            pl.BlockSpec((1, 2 * H), lambda i: (0, 0)),
            pl.BlockSpec((1, 2 * H), lambda i: (0, 0)),
            pl.BlockSpec((1, 2 * H), lambda i: (0, 0)),
            pl.BlockSpec((2 * H, H), lambda i: (0, 0)),
            pl.BlockSpec((1, H), lambda i: (0, 0)),
        ],
        out_specs=pl.BlockSpec((ROW_BLK, H), lambda i: (i, 0)),
        out_shape=jax.ShapeDtypeStruct((N, H), jnp.float32),
    )(O, O, z, h, w1, b1, mg, mb, w2, b2)


def _fin_body(h_ref, g_ref, b_ref, w_ref, bo_ref, out_ref):
    z = jnp.maximum(_ln(h_ref[...], g_ref[...], b_ref[...]), 0.0)
    out_ref[...] = jnp.sum(z * w_ref[...], axis=1, keepdims=True) + bo_ref[...]


def _final(h, g, b, wt, bo):
    return pl.pallas_call(
        _fin_body,
        grid=(NBLK,),
        in_specs=[
            pl.BlockSpec((ROW_BLK, H), lambda i: (i, 0)),
            pl.BlockSpec((1, H), lambda i: (0, 0)),
            pl.BlockSpec((1, H), lambda i: (0, 0)),
            pl.BlockSpec((1, H), lambda i: (0, 0)),
            pl.BlockSpec((1, 1), lambda i: (0, 0)),
        ],
        out_specs=pl.BlockSpec((ROW_BLK, 1), lambda i: (i, 0)),
        out_shape=jax.ShapeDtypeStruct((N, 1), jnp.float32),
    )(h, g, b, wt, bo)


_MESH = plsc.VectorSubcoreMesh(core_axis_name="c", subcore_axis_name="s")


@functools.partial(
    pl.kernel,
    mesh=_MESH,
    out_type=jax.ShapeDtypeStruct((2 * ACC_ROWS, H), jnp.float32),
    scratch_types=[
        pltpu.VMEM((GSTAGE, CHUNK), jnp.int32),
        pltpu.VMEM((GSTAGE, CHUNK), jnp.int32),
        pltpu.VMEM((CHUNK, H), jnp.float32),
        pltpu.VMEM_SHARED((ACC_ROWS, H), jnp.float32),
        pltpu.SemaphoreType.DMA,
    ],
)
def _edge_kernel(src_hbm, dst_hbm, t_hbm, z_hbm, o_hbm,
                 sidx_v, didx_v, rows_v, acc, sem):
    c = lax.axis_index("c")
    s = lax.axis_index("s")
    pltpu.sync_copy(z_hbm, acc.at[pl.ds(s * ROWS_PER_SUB, ROWS_PER_SUB)])
    plsc.subcore_barrier()

    def stage(j, carry):
        pltpu.sync_copy(src_hbm.at[c, s, pl.ds(j * GSTAGE, GSTAGE)], sidx_v)
        pltpu.sync_copy(dst_hbm.at[s, pl.ds(j * GSTAGE, GSTAGE)], didx_v)

        def chunk(i, carry2):
            pltpu.async_copy(t_hbm.at[sidx_v.at[i]], rows_v, sem).wait()
            pltpu.sync_copy(rows_v, acc.at[didx_v.at[i]], add=True)
            return carry2

        lax.fori_loop(0, GSTAGE, chunk, 0)
        return carry

    lax.fori_loop(0, NSTAGE, stage, 0)
    plsc.subcore_barrier()
    pltpu.sync_copy(
        acc.at[pl.ds(s * ROWS_PER_SUB, ROWS_PER_SUB)],
        o_hbm.at[pl.ds(c * ACC_ROWS + s * ROWS_PER_SUB, ROWS_PER_SUB)],
    )


def kernel(x, edge_index, W_enc, b_enc, ln_g, ln_b, t, W1, b1, mg, mb, W2,
           b2, W_out, b_out):
    src = edge_index[0]
    dst = edge_index[1]
    base = jnp.pad(src, (0, EPAD - E)).reshape(NSUB, NCHUNK, CHUNK)
    src4 = jnp.stack([base, base + N])
    dst3 = jnp.pad(dst, (0, EPAD - E), constant_values=N).reshape(
        NSUB, NCHUNK, CHUNK)
    zrows = jnp.zeros((ROWS_PER_SUB, H), jnp.float32)

    xp = jnp.pad(x, ((0, 0), (0, DPAD - DIN)))
    wp = jnp.pad(W_enc, ((0, DPAD - DIN), (0, 0)))
    h = _encoder(xp, wp, b_enc.reshape(1, H))

    L = t.shape[0]
    for i in range(L):
        tb = jnp.full((1, H), t[i], jnp.float32)
        z, gb = _prep1(h, ln_g[i].reshape(1, H), ln_b[i].reshape(1, H), tb,
                       first=(i == 0))
        T = _prep2(z, gb, tb)
        O = _edge_kernel(src4, dst3, T.reshape(2 * N, H), zrows)
        O = O.reshape(2, ACC_ROWS, H)
        h = _post(O, z, h, W1[i], b1[i].reshape(1, 2 * H),
                  mg[i].reshape(1, 2 * H), mb[i].reshape(1, 2 * H), W2[i],
                  b2[i].reshape(1, H), residual=(i > 0))

    return _final(h, ln_g[0].reshape(1, H), ln_b[0].reshape(1, H),
                  W_out.reshape(1, H), b_out.reshape(1, 1))


# speedup vs baseline: 5.8015x; 1.1024x over previous
import functools

import jax
import jax.numpy as jnp
from jax import lax
from jax.experimental import pallas as pl
from jax.experimental.pallas import tpu as pltpu
from jax.experimental.pallas import tpu_sc as plsc

N = 10000
E = 320000
H = 128
DIN = 1070
DPAD = 1152

NSUB = 16
CHUNK = 64
NCHUNK = 320
GSTAGE = 40
NSTAGE = NCHUNK // GSTAGE
NSLOT = 4
GRP = GSTAGE // NSLOT
EPAD = NSUB * NCHUNK * CHUNK
ROWS_PER_SUB = 632
ACC_ROWS = ROWS_PER_SUB * NSUB

ROW_BLK = 1000
NBLK = N // ROW_BLK


def _enc_body(x_ref, w_ref, b_ref, o_ref):
    o_ref[...] = (
        jnp.dot(x_ref[...], w_ref[...], preferred_element_type=jnp.float32)
        + b_ref[...]
    )


def _encoder(xp, wp, b):
    return pl.pallas_call(
        _enc_body,
        grid=(NBLK,),
        in_specs=[
            pl.BlockSpec((ROW_BLK, DPAD), lambda i: (i, 0)),
            pl.BlockSpec((DPAD, H), lambda i: (0, 0)),
            pl.BlockSpec((1, H), lambda i: (0, 0)),
        ],
        out_specs=pl.BlockSpec((ROW_BLK, H), lambda i: (i, 0)),
        out_shape=jax.ShapeDtypeStruct((N, H), jnp.float32),
    )(xp, wp, b)


def _ln(v, g, b):
    mu = jnp.mean(v, axis=-1, keepdims=True)
    var = jnp.mean((v - mu) ** 2, axis=-1, keepdims=True)
    return (v - mu) * lax.rsqrt(var + 1e-5) * g + b


def _prep1_body(h_ref, g_ref, b_ref, t_ref, z_ref, gb_ref, *, first):
    h = h_ref[...]
    if first:
        z = h
    else:
        z = jnp.maximum(_ln(h, g_ref[...], b_ref[...]), 0.0)
    z_ref[...] = z
    m = jnp.maximum(z, 0.0) + 1e-7
    gb_ref[...] = jnp.max(m * t_ref[...], axis=0, keepdims=True).reshape(1, 1, -1)


def _prep1(h, g, b, tb, first):
    return pl.pallas_call(
        functools.partial(_prep1_body, first=first),
        grid=(NBLK,),
        in_specs=[
            pl.BlockSpec((ROW_BLK, H), lambda i: (i, 0)),
            pl.BlockSpec((1, H), lambda i: (0, 0)),
            pl.BlockSpec((1, H), lambda i: (0, 0)),
            pl.BlockSpec((1, H), lambda i: (0, 0)),
        ],
        out_specs=[
            pl.BlockSpec((ROW_BLK, H), lambda i: (i, 0)),
            pl.BlockSpec((1, 1, H), lambda i: (i, 0, 0)),
        ],
        out_shape=[
            jax.ShapeDtypeStruct((N, H), jnp.float32),
            jax.ShapeDtypeStruct((NBLK, 1, H), jnp.float32),
        ],
    )(h, g, b, tb)


def _prep2_body(z_ref, gb_ref, t_ref, T_ref):
    z = z_ref[...]
    m = jnp.maximum(z, 0.0) + 1e-7
    logit = m * t_ref[...]
    G = jnp.max(gb_ref[...].reshape(-1, gb_ref.shape[-1]), axis=0, keepdims=True)
    P = jnp.exp(logit - G)
    Q = m * P
    T_ref[0, :, :64] = P[:, :64]
    T_ref[0, :, 64:] = Q[:, :64]
    T_ref[1, :, :64] = P[:, 64:]
    T_ref[1, :, 64:] = Q[:, 64:]


def _prep2(z, gb, tb):
    return pl.pallas_call(
        _prep2_body,
        grid=(NBLK,),
        in_specs=[
            pl.BlockSpec((ROW_BLK, H), lambda i: (i, 0)),
            pl.BlockSpec((NBLK, 1, H), lambda i: (0, 0, 0)),
            pl.BlockSpec((1, H), lambda i: (0, 0)),
        ],
        out_specs=pl.BlockSpec((2, ROW_BLK, H), lambda i: (0, i, 0)),
        out_shape=jax.ShapeDtypeStruct((2, N, H), jnp.float32),
    )(z, gb, tb)


def _post_body(o0_ref, o1_ref, z_ref, h_ref, w1_ref, b1_ref, mg_ref, mb_ref,
               w2_ref, b2_ref, out_ref, *, residual):
    den = jnp.concatenate([o0_ref[0, :, :64], o1_ref[0, :, :64]], axis=1)
    num = jnp.concatenate([o0_ref[0, :, 64:], o1_ref[0, :, 64:]], axis=1)
    aggr = num / (den + 1e-16)
    hm = aggr + z_ref[...]
    u = jnp.dot(hm, w1_ref[...], preferred_element_type=jnp.float32) + b1_ref[...]
    u = jnp.maximum(_ln(u, mg_ref[...], mb_ref[...]), 0.0)
    y = jnp.dot(u, w2_ref[...], preferred_element_type=jnp.float32) + b2_ref[...]
    if residual:
        y = y + h_ref[...]
    out_ref[...] = y


def _post(O, z, h, w1, b1, mg, mb, w2, b2, residual):
    return pl.pallas_call(
        functools.partial(_post_body, residual=residual),
        grid=(NBLK,),
        in_specs=[
            pl.BlockSpec((1, ROW_BLK, H), lambda i: (0, i, 0)),
            pl.BlockSpec((1, ROW_BLK, H), lambda i: (1, i, 0)),
            pl.BlockSpec((ROW_BLK, H), lambda i: (i, 0)),
            pl.BlockSpec((ROW_BLK, H), lambda i: (i, 0)),
            pl.BlockSpec((H, 2 * H), lambda i: (0, 0)),
            pl.BlockSpec((1, 2 * H), lambda i: (0, 0)),
            pl.BlockSpec((1, 2 * H), lambda i: (0, 0)),
            pl.BlockSpec((1, 2 * H), lambda i: (0, 0)),
            pl.BlockSpec((2 * H, H), lambda i: (0, 0)),
            pl.BlockSpec((1, H), lambda i: (0, 0)),
        ],
        out_specs=pl.BlockSpec((ROW_BLK, H), lambda i: (i, 0)),
        out_shape=jax.ShapeDtypeStruct((N, H), jnp.float32),
    )(O, O, z, h, w1, b1, mg, mb, w2, b2)


def _fin_body(h_ref, g_ref, b_ref, w_ref, bo_ref, out_ref):
    z = jnp.maximum(_ln(h_ref[...], g_ref[...], b_ref[...]), 0.0)
    out_ref[...] = jnp.sum(z * w_ref[...], axis=1, keepdims=True) + bo_ref[...]


def _final(h, g, b, wt, bo):
    return pl.pallas_call(
        _fin_body,
        grid=(NBLK,),
        in_specs=[
            pl.BlockSpec((ROW_BLK, H), lambda i: (i, 0)),
            pl.BlockSpec((1, H), lambda i: (0, 0)),
            pl.BlockSpec((1, H), lambda i: (0, 0)),
            pl.BlockSpec((1, H), lambda i: (0, 0)),
            pl.BlockSpec((1, 1), lambda i: (0, 0)),
        ],
        out_specs=pl.BlockSpec((ROW_BLK, 1), lambda i: (i, 0)),
        out_shape=jax.ShapeDtypeStruct((N, 1), jnp.float32),
    )(h, g, b, wt, bo)


_MESH = plsc.VectorSubcoreMesh(core_axis_name="c", subcore_axis_name="s")


@functools.partial(
    pl.kernel,
    mesh=_MESH,
    out_type=jax.ShapeDtypeStruct((2 * ACC_ROWS, H), jnp.float32),
    scratch_types=[
        pltpu.VMEM((GSTAGE, CHUNK), jnp.int32),
        pltpu.VMEM((GSTAGE, CHUNK), jnp.int32),
        pltpu.VMEM((NSLOT, CHUNK, H), jnp.float32),
        pltpu.VMEM_SHARED((ACC_ROWS, H), jnp.float32),
        pltpu.SemaphoreType.DMA((NSLOT,)),
        pltpu.SemaphoreType.DMA((NSLOT,)),
    ],
)
def _edge_kernel(src_hbm, dst_hbm, t_hbm, z_hbm, o_hbm,
                 sidx_v, didx_v, rows_v, acc, gsem, ssem):
    c = lax.axis_index("c")
    s = lax.axis_index("s")
    tbl = t_hbm.at[c]
    pltpu.sync_copy(z_hbm, acc.at[pl.ds(s * ROWS_PER_SUB, ROWS_PER_SUB)])
    plsc.subcore_barrier()

    def stage(g, carry):
        pltpu.sync_copy(src_hbm.at[s, pl.ds(g * GSTAGE, GSTAGE)], sidx_v)
        pltpu.sync_copy(dst_hbm.at[s, pl.ds(g * GSTAGE, GSTAGE)], didx_v)

        for b in range(NSLOT):
            pltpu.async_copy(tbl.at[sidx_v.at[b]], rows_v.at[b], gsem.at[b])

        def grp(k, carry2):
            for b in range(NSLOT):
                j = k * NSLOT + b
                pltpu.make_async_copy(
                    tbl.at[sidx_v.at[j]], rows_v.at[b], gsem.at[b]).wait()
                pltpu.async_copy(
                    rows_v.at[b], acc.at[didx_v.at[j]], ssem.at[b], add=True)

            @pl.when(k < GRP - 1)
            def _():
                for b in range(NSLOT):
                    j = (k + 1) * NSLOT + b
                    pltpu.make_async_copy(
                        rows_v.at[b], acc.at[didx_v.at[j - NSLOT]],
                        ssem.at[b]).wait()
                    pltpu.async_copy(
                        tbl.at[sidx_v.at[j]], rows_v.at[b], gsem.at[b])
            return carry2

        lax.fori_loop(0, GRP, grp, 0)
        for b in range(NSLOT):
            pltpu.make_async_copy(
                rows_v.at[b], acc.at[didx_v.at[(GRP - 1) * NSLOT + b]],
                ssem.at[b]).wait()
        return carry

    lax.fori_loop(0, NSTAGE, stage, 0)
    plsc.subcore_barrier()
    pltpu.sync_copy(
        acc.at[pl.ds(s * ROWS_PER_SUB, ROWS_PER_SUB)],
        o_hbm.at[pl.ds(c * ACC_ROWS + s * ROWS_PER_SUB, ROWS_PER_SUB)],
    )


def kernel(x, edge_index, W_enc, b_enc, ln_g, ln_b, t, W1, b1, mg, mb, W2,
           b2, W_out, b_out):
    src = edge_index[0]
    dst = edge_index[1]
    src3 = jnp.pad(src, (0, EPAD - E)).reshape(NSUB, NCHUNK, CHUNK)
    dst3 = jnp.pad(dst, (0, EPAD - E), constant_values=N).reshape(
        NSUB, NCHUNK, CHUNK)
    zrows = jnp.zeros((ROWS_PER_SUB, H), jnp.float32)

    xp = jnp.pad(x, ((0, 0), (0, DPAD - DIN)))
    wp = jnp.pad(W_enc, ((0, DPAD - DIN), (0, 0)))
    h = _encoder(xp, wp, b_enc.reshape(1, H))

    L = t.shape[0]
    for i in range(L):
        tb = jnp.full((1, H), t[i], jnp.float32)
        z, gb = _prep1(h, ln_g[i].reshape(1, H), ln_b[i].reshape(1, H), tb,
                       first=(i == 0))
        T = _prep2(z, gb, tb)
        O = _edge_kernel(src3, dst3, T, zrows)
        O = O.reshape(2, ACC_ROWS, H)
        h = _post(O, z, h, W1[i], b1[i].reshape(1, 2 * H),
                  mg[i].reshape(1, 2 * H), mb[i].reshape(1, 2 * H), W2[i],
                  b2[i].reshape(1, H), residual=(i > 0))

    return _final(h, ln_g[0].reshape(1, H), ln_b[0].reshape(1, H),
                  W_out.reshape(1, H), b_out.reshape(1, 1))


# speedup vs baseline: 6.6760x; 1.1507x over previous
import functools

import jax
import jax.numpy as jnp
from jax import lax
from jax.experimental import pallas as pl
from jax.experimental.pallas import tpu as pltpu
from jax.experimental.pallas import tpu_sc as plsc

N = 10000
E = 320000
H = 128
DIN = 1070
DPAD = 1152

NSUB = 16
CHUNK = 64
NCHUNK = 320
GSTAGE = 40
NSTAGE = NCHUNK // GSTAGE
NSLOT = 4
GRP = GSTAGE // NSLOT
EPAD = NSUB * NCHUNK * CHUNK
ROWS_PER_SUB = 632
ACC_ROWS = ROWS_PER_SUB * NSUB

ROW_BLK = 1000
NBLK = N // ROW_BLK


def _enc_body(x_ref, w_ref, b_ref, o_ref):
    o_ref[...] = (
        jnp.dot(x_ref[...], w_ref[...], preferred_element_type=jnp.float32)
        + b_ref[...]
    )


def _encoder(xp, wp, b):
    return pl.pallas_call(
        _enc_body,
        grid=(NBLK,),
        in_specs=[
            pl.BlockSpec((ROW_BLK, DPAD), lambda i: (i, 0)),
            pl.BlockSpec((DPAD, H), lambda i: (0, 0)),
            pl.BlockSpec((1, H), lambda i: (0, 0)),
        ],
        out_specs=pl.BlockSpec((ROW_BLK, H), lambda i: (i, 0)),
        out_shape=jax.ShapeDtypeStruct((N, H), jnp.float32),
    )(xp, wp, b)


def _ln(v, g, b):
    mu = jnp.mean(v, axis=-1, keepdims=True)
    var = jnp.mean((v - mu) ** 2, axis=-1, keepdims=True)
    return (v - mu) * lax.rsqrt(var + 1e-5) * g + b


def _prep1_body(h_ref, g_ref, b_ref, t_ref, z_ref, gb_ref, *, first):
    h = h_ref[...]
    if first:
        z = h
    else:
        z = jnp.maximum(_ln(h, g_ref[...], b_ref[...]), 0.0)
    z_ref[...] = z
    m = jnp.maximum(z, 0.0) + 1e-7
    gb_ref[...] = jnp.max(m * t_ref[...], axis=0, keepdims=True).reshape(1, 1, -1)


def _prep1(h, g, b, tb, first):
    return pl.pallas_call(
        functools.partial(_prep1_body, first=first),
        grid=(NBLK,),
        in_specs=[
            pl.BlockSpec((ROW_BLK, H), lambda i: (i, 0)),
            pl.BlockSpec((1, H), lambda i: (0, 0)),
            pl.BlockSpec((1, H), lambda i: (0, 0)),
            pl.BlockSpec((1, H), lambda i: (0, 0)),
        ],
        out_specs=[
            pl.BlockSpec((ROW_BLK, H), lambda i: (i, 0)),
            pl.BlockSpec((1, 1, H), lambda i: (i, 0, 0)),
        ],
        out_shape=[
            jax.ShapeDtypeStruct((N, H), jnp.float32),
            jax.ShapeDtypeStruct((NBLK, 1, H), jnp.float32),
        ],
    )(h, g, b, tb)


def _prep2_body(z_ref, gb_ref, t_ref, m_ref, aux_ref):
    z = z_ref[...]
    m = jnp.maximum(z, 0.0) + 1e-7
    G = jnp.max(gb_ref[...].reshape(-1, gb_ref.shape[-1]), axis=0, keepdims=True)
    m_ref[0, :, :] = m[:, :64]
    m_ref[1, :, :] = m[:, 64:]
    t16 = t_ref[0:1, 0:16]
    aux_ref[0:1, :] = jnp.concatenate([G[:, :64], t16], axis=1)
    aux_ref[1:2, :] = jnp.concatenate([G[:, 64:], t16], axis=1)


def _prep2(z, gb, tb):
    return pl.pallas_call(
        _prep2_body,
        grid=(NBLK,),
        in_specs=[
            pl.BlockSpec((ROW_BLK, H), lambda i: (i, 0)),
            pl.BlockSpec((NBLK, 1, H), lambda i: (0, 0, 0)),
            pl.BlockSpec((1, H), lambda i: (0, 0)),
        ],
        out_specs=[
            pl.BlockSpec((2, ROW_BLK, 64), lambda i: (0, i, 0)),
            pl.BlockSpec((2, 80), lambda i: (0, 0)),
        ],
        out_shape=[
            jax.ShapeDtypeStruct((2, N, 64), jnp.float32),
            jax.ShapeDtypeStruct((2, 80), jnp.float32),
        ],
    )(z, gb, tb)


def _post_body(pl_ref, ph_ref, ql_ref, qh_ref, z_ref, h_ref, w1_ref, b1_ref,
               mg_ref, mb_ref, w2_ref, b2_ref, out_ref, *, residual):
    den = jnp.concatenate([pl_ref[0], ph_ref[0]], axis=1)
    num = jnp.concatenate([ql_ref[0], qh_ref[0]], axis=1)
    aggr = num / (den + 1e-16)
    hm = aggr + z_ref[...]
    u = jnp.dot(hm, w1_ref[...], preferred_element_type=jnp.float32) + b1_ref[...]
    u = jnp.maximum(_ln(u, mg_ref[...], mb_ref[...]), 0.0)
    y = jnp.dot(u, w2_ref[...], preferred_element_type=jnp.float32) + b2_ref[...]
    if residual:
        y = y + h_ref[...]
    out_ref[...] = y


def _post(Op, Oq, z, h, w1, b1, mg, mb, w2, b2, residual):
    return pl.pallas_call(
        functools.partial(_post_body, residual=residual),
        grid=(NBLK,),
        in_specs=[
            pl.BlockSpec((1, ROW_BLK, 64), lambda i: (0, i, 0)),
            pl.BlockSpec((1, ROW_BLK, 64), lambda i: (1, i, 0)),
            pl.BlockSpec((1, ROW_BLK, 64), lambda i: (0, i, 0)),
            pl.BlockSpec((1, ROW_BLK, 64), lambda i: (1, i, 0)),
            pl.BlockSpec((ROW_BLK, H), lambda i: (i, 0)),
            pl.BlockSpec((ROW_BLK, H), lambda i: (i, 0)),
            pl.BlockSpec((H, 2 * H), lambda i: (0, 0)),
            pl.BlockSpec((1, 2 * H), lambda i: (0, 0)),
            pl.BlockSpec((1, 2 * H), lambda i: (0, 0)),
            pl.BlockSpec((1, 2 * H), lambda i: (0, 0)),
            pl.BlockSpec((2 * H, H), lambda i: (0, 0)),
            pl.BlockSpec((1, H), lambda i: (0, 0)),
        ],
        out_specs=pl.BlockSpec((ROW_BLK, H), lambda i: (i, 0)),
        out_shape=jax.ShapeDtypeStruct((N, H), jnp.float32),
    )(Op, Op, Oq, Oq, z, h, w1, b1, mg, mb, w2, b2)


def _fin_body(h_ref, g_ref, b_ref, w_ref, bo_ref, out_ref):
    z = jnp.maximum(_ln(h_ref[...], g_ref[...], b_ref[...]), 0.0)
    out_ref[...] = jnp.sum(z * w_ref[...], axis=1, keepdims=True) + bo_ref[...]


def _final(h, g, b, wt, bo):
    return pl.pallas_call(
        _fin_body,
        grid=(NBLK,),
        in_specs=[
            pl.BlockSpec((ROW_BLK, H), lambda i: (i, 0)),
            pl.BlockSpec((1, H), lambda i: (0, 0)),
            pl.BlockSpec((1, H), lambda i: (0, 0)),
            pl.BlockSpec((1, H), lambda i: (0, 0)),
            pl.BlockSpec((1, 1), lambda i: (0, 0)),
        ],
        out_specs=pl.BlockSpec((ROW_BLK, 1), lambda i: (i, 0)),
        out_shape=jax.ShapeDtypeStruct((N, 1), jnp.float32),
    )(h, g, b, wt, bo)


_MESH = plsc.VectorSubcoreMesh(core_axis_name="c", subcore_axis_name="s")


@functools.partial(
    pl.kernel,
    mesh=_MESH,
    compiler_params=pltpu.CompilerParams(use_tc_tiling_on_sc=False),
    out_type=(
        jax.ShapeDtypeStruct((2 * ACC_ROWS, 64), jnp.float32),
        jax.ShapeDtypeStruct((2 * ACC_ROWS, 64), jnp.float32),
    ),
    scratch_types=[
        pltpu.VMEM((GSTAGE, CHUNK), jnp.int32),
        pltpu.VMEM((GSTAGE, CHUNK), jnp.int32),
        pltpu.VMEM((NSLOT, CHUNK, 64), jnp.float32),
        pltpu.VMEM((NSLOT, CHUNK, 64), jnp.float32),
        pltpu.VMEM((80,), jnp.float32),
        pltpu.VMEM_SHARED((ACC_ROWS, 64), jnp.float32),
        pltpu.VMEM_SHARED((ACC_ROWS, 64), jnp.float32),
        pltpu.SemaphoreType.DMA((NSLOT,)),
        pltpu.SemaphoreType.DMA((NSLOT,)),
        pltpu.SemaphoreType.DMA((NSLOT,)),
    ],
)
def _edge_kernel(src_hbm, dst_hbm, m_hbm, aux_hbm, z_hbm, op_hbm, oq_hbm,
                 sidx_v, didx_v, rows_p, rows_q, aux_v, acc_p, acc_q,
                 gsem, psem, qsem):
    c = lax.axis_index("c")
    s = lax.axis_index("s")
    tbl = m_hbm.at[c]
    pltpu.sync_copy(aux_hbm.at[c], aux_v)
    gvec = [aux_v[pl.ds(16 * k, 16)] for k in range(4)]
    tvec = aux_v[pl.ds(64, 16)]
    pltpu.sync_copy(z_hbm, acc_p.at[pl.ds(s * ROWS_PER_SUB, ROWS_PER_SUB)])
    pltpu.sync_copy(z_hbm, acc_q.at[pl.ds(s * ROWS_PER_SUB, ROWS_PER_SUB)])
    plsc.subcore_barrier()

    def compute(b):
        R = rows_p.at[b]
        Qr = rows_q.at[b]

        def edge(e, cc):
            for k in range(4):
                mk = R[e, pl.ds(16 * k, 16)]
                p = jnp.exp(mk * tvec - gvec[k])
                R[e, pl.ds(16 * k, 16)] = p
                Qr[e, pl.ds(16 * k, 16)] = mk * p
            return cc

        lax.fori_loop(0, CHUNK, edge, 0)

    def stage(g, carry):
        pltpu.sync_copy(src_hbm.at[s, pl.ds(g * GSTAGE, GSTAGE)], sidx_v)
        pltpu.sync_copy(dst_hbm.at[s, pl.ds(g * GSTAGE, GSTAGE)], didx_v)

        for b in range(NSLOT):
            pltpu.async_copy(tbl.at[sidx_v.at[b]], rows_p.at[b], gsem.at[b])

        def grp(k, carry2):
            for b in range(NSLOT):
                j = k * NSLOT + b
                pltpu.make_async_copy(
                    tbl.at[sidx_v.at[j]], rows_p.at[b], gsem.at[b]).wait()
                compute(b)
                pltpu.async_copy(
                    rows_p.at[b], acc_p.at[didx_v.at[j]], psem.at[b],
                    add=True)
                pltpu.async_copy(
                    rows_q.at[b], acc_q.at[didx_v.at[j]], qsem.at[b],
                    add=True)

            @pl.when(k < GRP - 1)
            def _():
                for b in range(NSLOT):
                    j = (k + 1) * NSLOT + b
                    pltpu.make_async_copy(
                        rows_p.at[b], acc_p.at[didx_v.at[j - NSLOT]],
                        psem.at[b]).wait()
                    pltpu.make_async_copy(
                        rows_q.at[b], acc_q.at[didx_v.at[j - NSLOT]],
                        qsem.at[b]).wait()
                    pltpu.async_copy(
                        tbl.at[sidx_v.at[j]], rows_p.at[b], gsem.at[b])
            return carry2

        lax.fori_loop(0, GRP, grp, 0)
        for b in range(NSLOT):
            j = (GRP - 1) * NSLOT + b
            pltpu.make_async_copy(
                rows_p.at[b], acc_p.at[didx_v.at[j]], psem.at[b]).wait()
            pltpu.make_async_copy(
                rows_q.at[b], acc_q.at[didx_v.at[j]], qsem.at[b]).wait()
        return carry

    lax.fori_loop(0, NSTAGE, stage, 0)
    plsc.subcore_barrier()
    pltpu.sync_copy(
        acc_p.at[pl.ds(s * ROWS_PER_SUB, ROWS_PER_SUB)],
        op_hbm.at[pl.ds(c * ACC_ROWS + s * ROWS_PER_SUB, ROWS_PER_SUB)],
    )
    pltpu.sync_copy(
        acc_q.at[pl.ds(s * ROWS_PER_SUB, ROWS_PER_SUB)],
        oq_hbm.at[pl.ds(c * ACC_ROWS + s * ROWS_PER_SUB, ROWS_PER_SUB)],
    )


def kernel(x, edge_index, W_enc, b_enc, ln_g, ln_b, t, W1, b1, mg, mb, W2,
           b2, W_out, b_out):
    src = edge_index[0]
    dst = edge_index[1]
    src3 = jnp.pad(src, (0, EPAD - E)).reshape(NSUB, NCHUNK, CHUNK)
    dst3 = jnp.pad(dst, (0, EPAD - E), constant_values=N).reshape(
        NSUB, NCHUNK, CHUNK)
    zrows = jnp.zeros((ROWS_PER_SUB, 64), jnp.float32)

    xp = jnp.pad(x, ((0, 0), (0, DPAD - DIN)))
    wp = jnp.pad(W_enc, ((0, DPAD - DIN), (0, 0)))
    h = _encoder(xp, wp, b_enc.reshape(1, H))

    L = t.shape[0]
    for i in range(L):
        tb = jnp.full((1, H), t[i], jnp.float32)
        z, gb = _prep1(h, ln_g[i].reshape(1, H), ln_b[i].reshape(1, H), tb,
                       first=(i == 0))
        M, AUX = _prep2(z, gb, tb)
        Op, Oq = _edge_kernel(src3, dst3, M, AUX, zrows)
        Op = Op.reshape(2, ACC_ROWS, 64)
        Oq = Oq.reshape(2, ACC_ROWS, 64)
        h = _post(Op, Oq, z, h, W1[i], b1[i].reshape(1, 2 * H),
                  mg[i].reshape(1, 2 * H), mb[i].reshape(1, 2 * H), W2[i],
                  b2[i].reshape(1, H), residual=(i > 0))

    return _final(h, ln_g[0].reshape(1, H), ln_b[0].reshape(1, H),
                  W_out.reshape(1, H), b_out.reshape(1, 1))


# speedup vs baseline: 8.0391x; 1.2042x over previous
import functools

import jax
import jax.numpy as jnp
from jax import lax
from jax.experimental import pallas as pl
from jax.experimental.pallas import tpu as pltpu
from jax.experimental.pallas import tpu_sc as plsc

N = 10000
E = 320000
H = 128
DIN = 1070
DPAD = 1152

NSUB = 16
CHUNK = 64
NCHUNK = 320
GSTAGE = 40
NSTAGE = NCHUNK // GSTAGE
NSLOT = 4
GRP = GSTAGE // NSLOT
EPAD = NSUB * NCHUNK * CHUNK
ROWS_PER_SUB = 632
ACC_ROWS = ROWS_PER_SUB * NSUB

ROW_BLK = 1000
NBLK = N // ROW_BLK


def _enc_body(x_ref, w_ref, b_ref, o_ref):
    o_ref[...] = (
        jnp.dot(x_ref[...], w_ref[...], preferred_element_type=jnp.float32)
        + b_ref[...]
    )


def _encoder(xp, wp, b):
    return pl.pallas_call(
        _enc_body,
        grid=(NBLK,),
        in_specs=[
            pl.BlockSpec((ROW_BLK, DPAD), lambda i: (i, 0)),
            pl.BlockSpec((DPAD, H), lambda i: (0, 0)),
            pl.BlockSpec((1, H), lambda i: (0, 0)),
        ],
        out_specs=pl.BlockSpec((ROW_BLK, H), lambda i: (i, 0)),
        out_shape=jax.ShapeDtypeStruct((N, H), jnp.float32),
    )(xp, wp, b)


def _ln(v, g, b):
    mu = jnp.mean(v, axis=-1, keepdims=True)
    var = jnp.mean((v - mu) ** 2, axis=-1, keepdims=True)
    return (v - mu) * lax.rsqrt(var + 1e-5) * g + b


def _prep1_body(h_ref, g_ref, b_ref, t_ref, z_ref, gb_ref, *, first):
    h = h_ref[...]
    if first:
        z = h
    else:
        z = jnp.maximum(_ln(h, g_ref[...], b_ref[...]), 0.0)
    z_ref[...] = z
    m = jnp.maximum(z, 0.0) + 1e-7
    gb_ref[...] = jnp.max(m * t_ref[...], axis=0, keepdims=True).reshape(1, 1, -1)


def _prep1(h, g, b, tb, first):
    return pl.pallas_call(
        functools.partial(_prep1_body, first=first),
        grid=(NBLK,),
        in_specs=[
            pl.BlockSpec((ROW_BLK, H), lambda i: (i, 0)),
            pl.BlockSpec((1, H), lambda i: (0, 0)),
            pl.BlockSpec((1, H), lambda i: (0, 0)),
            pl.BlockSpec((1, H), lambda i: (0, 0)),
        ],
        out_specs=[
            pl.BlockSpec((ROW_BLK, H), lambda i: (i, 0)),
            pl.BlockSpec((1, 1, H), lambda i: (i, 0, 0)),
        ],
        out_shape=[
            jax.ShapeDtypeStruct((N, H), jnp.float32),
            jax.ShapeDtypeStruct((NBLK, 1, H), jnp.float32),
        ],
    )(h, g, b, tb)


def _prep2_body(z_ref, gb_ref, t_ref, m_ref, aux_ref):
    z = z_ref[...]
    m = jnp.maximum(z, 0.0) + 1e-7
    G = jnp.max(gb_ref[...].reshape(-1, gb_ref.shape[-1]), axis=0, keepdims=True)
    m_ref[0, :, :] = m[:, :64]
    m_ref[1, :, :] = m[:, 64:]
    t16 = t_ref[0:1, 0:16]
    aux_ref[0:1, :] = jnp.concatenate([G[:, :64], t16], axis=1)
    aux_ref[1:2, :] = jnp.concatenate([G[:, 64:], t16], axis=1)


def _prep2(z, gb, tb):
    return pl.pallas_call(
        _prep2_body,
        grid=(NBLK,),
        in_specs=[
            pl.BlockSpec((ROW_BLK, H), lambda i: (i, 0)),
            pl.BlockSpec((NBLK, 1, H), lambda i: (0, 0, 0)),
            pl.BlockSpec((1, H), lambda i: (0, 0)),
        ],
        out_specs=[
            pl.BlockSpec((2, ROW_BLK, 64), lambda i: (0, i, 0)),
            pl.BlockSpec((2, 80), lambda i: (0, 0)),
        ],
        out_shape=[
            jax.ShapeDtypeStruct((2, N, 64), jnp.float32),
            jax.ShapeDtypeStruct((2, 80), jnp.float32),
        ],
    )(z, gb, tb)


def _post_body(pl_ref, ph_ref, ql_ref, qh_ref, z_ref, h_ref, w1_ref, b1_ref,
               mg_ref, mb_ref, w2_ref, b2_ref, out_ref, *, residual):
    den = jnp.concatenate([pl_ref[0], ph_ref[0]], axis=1)
    num = jnp.concatenate([ql_ref[0], qh_ref[0]], axis=1)
    aggr = num / (den + 1e-16)
    hm = aggr + z_ref[...]
    u = jnp.dot(hm, w1_ref[...], preferred_element_type=jnp.float32) + b1_ref[...]
    u = jnp.maximum(_ln(u, mg_ref[...], mb_ref[...]), 0.0)
    y = jnp.dot(u, w2_ref[...], preferred_element_type=jnp.float32) + b2_ref[...]
    if residual:
        y = y + h_ref[...]
    out_ref[...] = y


def _post(Op, Oq, z, h, w1, b1, mg, mb, w2, b2, residual):
    return pl.pallas_call(
        functools.partial(_post_body, residual=residual),
        grid=(NBLK,),
        in_specs=[
            pl.BlockSpec((1, ROW_BLK, 64), lambda i: (0, i, 0)),
            pl.BlockSpec((1, ROW_BLK, 64), lambda i: (1, i, 0)),
            pl.BlockSpec((1, ROW_BLK, 64), lambda i: (0, i, 0)),
            pl.BlockSpec((1, ROW_BLK, 64), lambda i: (1, i, 0)),
            pl.BlockSpec((ROW_BLK, H), lambda i: (i, 0)),
            pl.BlockSpec((ROW_BLK, H), lambda i: (i, 0)),
            pl.BlockSpec((H, 2 * H), lambda i: (0, 0)),
            pl.BlockSpec((1, 2 * H), lambda i: (0, 0)),
            pl.BlockSpec((1, 2 * H), lambda i: (0, 0)),
            pl.BlockSpec((1, 2 * H), lambda i: (0, 0)),
            pl.BlockSpec((2 * H, H), lambda i: (0, 0)),
            pl.BlockSpec((1, H), lambda i: (0, 0)),
        ],
        out_specs=pl.BlockSpec((ROW_BLK, H), lambda i: (i, 0)),
        out_shape=jax.ShapeDtypeStruct((N, H), jnp.float32),
    )(Op, Op, Oq, Oq, z, h, w1, b1, mg, mb, w2, b2)


def _fin_body(h_ref, g_ref, b_ref, w_ref, bo_ref, out_ref):
    z = jnp.maximum(_ln(h_ref[...], g_ref[...], b_ref[...]), 0.0)
    out_ref[...] = jnp.sum(z * w_ref[...], axis=1, keepdims=True) + bo_ref[...]


def _final(h, g, b, wt, bo):
    return pl.pallas_call(
        _fin_body,
        grid=(NBLK,),
        in_specs=[
            pl.BlockSpec((ROW_BLK, H), lambda i: (i, 0)),
            pl.BlockSpec((1, H), lambda i: (0, 0)),
            pl.BlockSpec((1, H), lambda i: (0, 0)),
            pl.BlockSpec((1, H), lambda i: (0, 0)),
            pl.BlockSpec((1, 1), lambda i: (0, 0)),
        ],
        out_specs=pl.BlockSpec((ROW_BLK, 1), lambda i: (i, 0)),
        out_shape=jax.ShapeDtypeStruct((N, 1), jnp.float32),
    )(h, g, b, wt, bo)


_MESH = plsc.VectorSubcoreMesh(core_axis_name="c", subcore_axis_name="s")


@functools.partial(
    pl.kernel,
    mesh=_MESH,
    compiler_params=pltpu.CompilerParams(use_tc_tiling_on_sc=False),
    out_type=(
        jax.ShapeDtypeStruct((2 * ACC_ROWS, 64), jnp.float32),
        jax.ShapeDtypeStruct((2 * ACC_ROWS, 64), jnp.float32),
    ),
    scratch_types=[
        pltpu.VMEM((GSTAGE, CHUNK), jnp.int32),
        pltpu.VMEM((GSTAGE, CHUNK), jnp.int32),
        pltpu.VMEM((NSLOT, CHUNK, 64), jnp.float32),
        pltpu.VMEM((NSLOT, CHUNK, 64), jnp.float32),
        pltpu.VMEM((80,), jnp.float32),
        pltpu.VMEM_SHARED((ACC_ROWS, 64), jnp.float32),
        pltpu.VMEM_SHARED((ACC_ROWS, 64), jnp.float32),
        pltpu.SemaphoreType.DMA((NSLOT,)),
        pltpu.SemaphoreType.DMA((NSLOT,)),
        pltpu.SemaphoreType.DMA((NSLOT,)),
    ],
)
def _edge_kernel(src_hbm, dst_hbm, m_hbm, aux_hbm, z_hbm, op_hbm, oq_hbm,
                 sidx_v, didx_v, rows_p, rows_q, aux_v, acc_p, acc_q,
                 gsem, psem, qsem):
    c = lax.axis_index("c")
    s = lax.axis_index("s")
    tbl = m_hbm.at[c]
    pltpu.sync_copy(aux_hbm.at[c], aux_v)
    gvec = [aux_v[pl.ds(16 * k, 16)] for k in range(4)]
    tvec = aux_v[pl.ds(64, 16)]
    pltpu.sync_copy(z_hbm, acc_p.at[pl.ds(s * ROWS_PER_SUB, ROWS_PER_SUB)])
    pltpu.sync_copy(z_hbm, acc_q.at[pl.ds(s * ROWS_PER_SUB, ROWS_PER_SUB)])
    plsc.subcore_barrier()

    def compute(b):
        R = rows_p.at[b]
        Qr = rows_q.at[b]

        def edge(e4, cc):
            for u in range(4):
                e = e4 * 4 + u
                for k in range(4):
                    mk = R[e, pl.ds(16 * k, 16)]
                    p = jnp.exp(mk * tvec - gvec[k])
                    R[e, pl.ds(16 * k, 16)] = p
                    Qr[e, pl.ds(16 * k, 16)] = mk * p
            return cc

        lax.fori_loop(0, CHUNK // 4, edge, 0)

    def stage(g, carry):
        pltpu.sync_copy(src_hbm.at[s, pl.ds(g * GSTAGE, GSTAGE)], sidx_v)
        pltpu.sync_copy(dst_hbm.at[s, pl.ds(g * GSTAGE, GSTAGE)], didx_v)

        for b in range(NSLOT):
            pltpu.async_copy(tbl.at[sidx_v.at[b]], rows_p.at[b], gsem.at[b])

        def grp(k, carry2):
            for b in range(NSLOT):
                j = k * NSLOT + b
                pltpu.make_async_copy(
                    tbl.at[sidx_v.at[j]], rows_p.at[b], gsem.at[b]).wait()
                compute(b)
                pltpu.async_copy(
                    rows_p.at[b], acc_p.at[didx_v.at[j]], psem.at[b],
                    add=True)
                pltpu.async_copy(
                    rows_q.at[b], acc_q.at[didx_v.at[j]], qsem.at[b],
                    add=True)

            @pl.when(k < GRP - 1)
            def _():
                for b in range(NSLOT):
                    j = (k + 1) * NSLOT + b
                    pltpu.make_async_copy(
                        rows_p.at[b], acc_p.at[didx_v.at[j - NSLOT]],
                        psem.at[b]).wait()
                    pltpu.make_async_copy(
                        rows_q.at[b], acc_q.at[didx_v.at[j - NSLOT]],
                        qsem.at[b]).wait()
                    pltpu.async_copy(
                        tbl.at[sidx_v.at[j]], rows_p.at[b], gsem.at[b])
            return carry2

        lax.fori_loop(0, GRP, grp, 0)
        for b in range(NSLOT):
            j = (GRP - 1) * NSLOT + b
            pltpu.make_async_copy(
                rows_p.at[b], acc_p.at[didx_v.at[j]], psem.at[b]).wait()
            pltpu.make_async_copy(
                rows_q.at[b], acc_q.at[didx_v.at[j]], qsem.at[b]).wait()
        return carry

    lax.fori_loop(0, NSTAGE, stage, 0)
    plsc.subcore_barrier()
    pltpu.sync_copy(
        acc_p.at[pl.ds(s * ROWS_PER_SUB, ROWS_PER_SUB)],
        op_hbm.at[pl.ds(c * ACC_ROWS + s * ROWS_PER_SUB, ROWS_PER_SUB)],
    )
    pltpu.sync_copy(
        acc_q.at[pl.ds(s * ROWS_PER_SUB, ROWS_PER_SUB)],
        oq_hbm.at[pl.ds(c * ACC_ROWS + s * ROWS_PER_SUB, ROWS_PER_SUB)],
    )


def kernel(x, edge_index, W_enc, b_enc, ln_g, ln_b, t, W1, b1, mg, mb, W2,
           b2, W_out, b_out):
    src = edge_index[0]
    dst = edge_index[1]
    src3 = jnp.pad(src, (0, EPAD - E)).reshape(NSUB, NCHUNK, CHUNK)
    dst3 = jnp.pad(dst, (0, EPAD - E), constant_values=N).reshape(
        NSUB, NCHUNK, CHUNK)
    zrows = jnp.zeros((ROWS_PER_SUB, 64), jnp.float32)

    xp = jnp.pad(x, ((0, 0), (0, DPAD - DIN)))
    wp = jnp.pad(W_enc, ((0, DPAD - DIN), (0, 0)))
    h = _encoder(xp, wp, b_enc.reshape(1, H))

    L = t.shape[0]
    for i in range(L):
        tb = jnp.full((1, H), t[i], jnp.float32)
        z, gb = _prep1(h, ln_g[i].reshape(1, H), ln_b[i].reshape(1, H), tb,
                       first=(i == 0))
        M, AUX = _prep2(z, gb, tb)
        Op, Oq = _edge_kernel(src3, dst3, M, AUX, zrows)
        Op = Op.reshape(2, ACC_ROWS, 64)
        Oq = Oq.reshape(2, ACC_ROWS, 64)
        h = _post(Op, Oq, z, h, W1[i], b1[i].reshape(1, 2 * H),
                  mg[i].reshape(1, 2 * H), mb[i].reshape(1, 2 * H), W2[i],
                  b2[i].reshape(1, H), residual=(i > 0))

    return _final(h, ln_g[0].reshape(1, H), ln_b[0].reshape(1, H),
                  W_out.reshape(1, H), b_out.reshape(1, 1))


# speedup vs baseline: 11.9723x; 1.4893x over previous
import functools

import jax
import jax.numpy as jnp
from jax import lax
from jax.experimental import pallas as pl
from jax.experimental.pallas import tpu as pltpu
from jax.experimental.pallas import tpu_sc as plsc

N = 10000
E = 320000
H = 128
DIN = 1070
DPAD = 1152

NSUB = 16
CHUNK = 100
NCHUNK = 200
GSTAGE = 40
NSTAGE = NCHUNK // GSTAGE
NSLOT = 2
GRP = GSTAGE // NSLOT
ROWS_PER_SUB = 632
ACC_ROWS = ROWS_PER_SUB * NSUB

ROW_BLK = 1000
NBLK = N // ROW_BLK


def _enc_body(x_ref, w_ref, b_ref, o_ref):
    o_ref[...] = (
        jnp.dot(x_ref[...], w_ref[...], preferred_element_type=jnp.float32)
        + b_ref[...]
    )


def _encoder(xp, wp, b):
    return pl.pallas_call(
        _enc_body,
        grid=(NBLK,),
        in_specs=[
            pl.BlockSpec((ROW_BLK, DPAD), lambda i: (i, 0)),
            pl.BlockSpec((DPAD, H), lambda i: (0, 0)),
            pl.BlockSpec((1, H), lambda i: (0, 0)),
        ],
        out_specs=pl.BlockSpec((ROW_BLK, H), lambda i: (i, 0)),
        out_shape=jax.ShapeDtypeStruct((N, H), jnp.float32),
    )(xp, wp, b)


def _ln(v, g, b):
    mu = jnp.mean(v, axis=-1, keepdims=True)
    var = jnp.mean((v - mu) ** 2, axis=-1, keepdims=True)
    return (v - mu) * lax.rsqrt(var + 1e-5) * g + b


def _prep1_body(h_ref, g_ref, b_ref, t_ref, z_ref, gb_ref, *, first):
    h = h_ref[...]
    if first:
        z = h
    else:
        z = jnp.maximum(_ln(h, g_ref[...], b_ref[...]), 0.0)
    z_ref[...] = z
    m = jnp.maximum(z, 0.0) + 1e-7
    gb_ref[...] = jnp.max(m * t_ref[...], axis=0, keepdims=True).reshape(1, 1, -1)


def _prep1(h, g, b, tb, first):
    return pl.pallas_call(
        functools.partial(_prep1_body, first=first),
        grid=(NBLK,),
        in_specs=[
            pl.BlockSpec((ROW_BLK, H), lambda i: (i, 0)),
            pl.BlockSpec((1, H), lambda i: (0, 0)),
            pl.BlockSpec((1, H), lambda i: (0, 0)),
            pl.BlockSpec((1, H), lambda i: (0, 0)),
        ],
        out_specs=[
            pl.BlockSpec((ROW_BLK, H), lambda i: (i, 0)),
            pl.BlockSpec((1, 1, H), lambda i: (i, 0, 0)),
        ],
        out_shape=[
            jax.ShapeDtypeStruct((N, H), jnp.float32),
            jax.ShapeDtypeStruct((NBLK, 1, H), jnp.float32),
        ],
    )(h, g, b, tb)


def _prep2_body(z_ref, gb_ref, t_ref, m_ref, aux_ref):
    z = z_ref[...]
    m = jnp.maximum(z, 0.0) + 1e-7
    G = jnp.max(gb_ref[...].reshape(-1, gb_ref.shape[-1]), axis=0, keepdims=True)
    m_ref[0, :, :] = m[:, :64]
    m_ref[1, :, :] = m[:, 64:]
    t16 = t_ref[0:1, 0:16]
    aux_ref[0:1, :] = jnp.concatenate([G[:, :64], t16], axis=1)
    aux_ref[1:2, :] = jnp.concatenate([G[:, 64:], t16], axis=1)


def _prep2(z, gb, tb):
    return pl.pallas_call(
        _prep2_body,
        grid=(NBLK,),
        in_specs=[
            pl.BlockSpec((ROW_BLK, H), lambda i: (i, 0)),
            pl.BlockSpec((NBLK, 1, H), lambda i: (0, 0, 0)),
            pl.BlockSpec((1, H), lambda i: (0, 0)),
        ],
        out_specs=[
            pl.BlockSpec((2, ROW_BLK, 64), lambda i: (0, i, 0)),
            pl.BlockSpec((2, 80), lambda i: (0, 0)),
        ],
        out_shape=[
            jax.ShapeDtypeStruct((2, N, 64), jnp.float32),
            jax.ShapeDtypeStruct((2, 80), jnp.float32),
        ],
    )(z, gb, tb)


def _post_body(pl_ref, ph_ref, ql_ref, qh_ref, z_ref, h_ref, w1_ref, b1_ref,
               mg_ref, mb_ref, w2_ref, b2_ref, out_ref, *, residual):
    den = jnp.concatenate([pl_ref[0], ph_ref[0]], axis=1)
    num = jnp.concatenate([ql_ref[0], qh_ref[0]], axis=1)
    aggr = num / (den + 1e-16)
    hm = aggr + z_ref[...]
    u = jnp.dot(hm, w1_ref[...], preferred_element_type=jnp.float32) + b1_ref[...]
    u = jnp.maximum(_ln(u, mg_ref[...], mb_ref[...]), 0.0)
    y = jnp.dot(u, w2_ref[...], preferred_element_type=jnp.float32) + b2_ref[...]
    if residual:
        y = y + h_ref[...]
    out_ref[...] = y


def _post(Op, Oq, z, h, w1, b1, mg, mb, w2, b2, residual):
    return pl.pallas_call(
        functools.partial(_post_body, residual=residual),
        grid=(NBLK,),
        in_specs=[
            pl.BlockSpec((1, ROW_BLK, 64), lambda i: (0, i, 0)),
            pl.BlockSpec((1, ROW_BLK, 64), lambda i: (1, i, 0)),
            pl.BlockSpec((1, ROW_BLK, 64), lambda i: (0, i, 0)),
            pl.BlockSpec((1, ROW_BLK, 64), lambda i: (1, i, 0)),
            pl.BlockSpec((ROW_BLK, H), lambda i: (i, 0)),
            pl.BlockSpec((ROW_BLK, H), lambda i: (i, 0)),
            pl.BlockSpec((H, 2 * H), lambda i: (0, 0)),
            pl.BlockSpec((1, 2 * H), lambda i: (0, 0)),
            pl.BlockSpec((1, 2 * H), lambda i: (0, 0)),
            pl.BlockSpec((1, 2 * H), lambda i: (0, 0)),
            pl.BlockSpec((2 * H, H), lambda i: (0, 0)),
            pl.BlockSpec((1, H), lambda i: (0, 0)),
        ],
        out_specs=pl.BlockSpec((ROW_BLK, H), lambda i: (i, 0)),
        out_shape=jax.ShapeDtypeStruct((N, H), jnp.float32),
    )(Op, Op, Oq, Oq, z, h, w1, b1, mg, mb, w2, b2)


def _fin_body(h_ref, g_ref, b_ref, w_ref, bo_ref, out_ref):
    z = jnp.maximum(_ln(h_ref[...], g_ref[...], b_ref[...]), 0.0)
    out_ref[...] = jnp.sum(z * w_ref[...], axis=1, keepdims=True) + bo_ref[...]


def _final(h, g, b, wt, bo):
    return pl.pallas_call(
        _fin_body,
        grid=(NBLK,),
        in_specs=[
            pl.BlockSpec((ROW_BLK, H), lambda i: (i, 0)),
            pl.BlockSpec((1, H), lambda i: (0, 0)),
            pl.BlockSpec((1, H), lambda i: (0, 0)),
            pl.BlockSpec((1, H), lambda i: (0, 0)),
            pl.BlockSpec((1, 1), lambda i: (0, 0)),
        ],
        out_specs=pl.BlockSpec((ROW_BLK, 1), lambda i: (i, 0)),
        out_shape=jax.ShapeDtypeStruct((N, 1), jnp.float32),
    )(h, g, b, wt, bo)


_MESH = plsc.VectorSubcoreMesh(core_axis_name="c", subcore_axis_name="s")


@functools.partial(
    pl.kernel,
    mesh=_MESH,
    compiler_params=pltpu.CompilerParams(use_tc_tiling_on_sc=False),
    out_type=(
        jax.ShapeDtypeStruct((2 * ACC_ROWS, 64), jnp.float32),
        jax.ShapeDtypeStruct((2 * ACC_ROWS, 64), jnp.float32),
    ),
    scratch_types=[
        pltpu.VMEM((GSTAGE, CHUNK), jnp.int32),
        pltpu.VMEM((GSTAGE, CHUNK), jnp.int32),
        pltpu.VMEM((NSLOT, CHUNK, 64), jnp.float32),
        pltpu.VMEM((NSLOT, CHUNK, 64), jnp.float32),
        pltpu.VMEM((80,), jnp.float32),
        pltpu.VMEM_SHARED((ACC_ROWS, 64), jnp.float32),
        pltpu.VMEM_SHARED((ACC_ROWS, 64), jnp.float32),
        pltpu.SemaphoreType.DMA((NSLOT,)),
        pltpu.SemaphoreType.DMA((NSLOT,)),
        pltpu.SemaphoreType.DMA((NSLOT,)),
    ],
)
def _edge_kernel(src_hbm, dst_hbm, m_hbm, aux_hbm, z_hbm, op_hbm, oq_hbm,
                 sidx_v, didx_v, rows_p, rows_q, aux_v, acc_p, acc_q,
                 gsem, psem, qsem):
    c = lax.axis_index("c")
    s = lax.axis_index("s")
    tbl = m_hbm.at[c]
    pltpu.sync_copy(aux_hbm.at[c], aux_v)
    gvec = [aux_v[pl.ds(16 * k, 16)] for k in range(4)]
    tvec = aux_v[pl.ds(64, 16)]
    pltpu.sync_copy(z_hbm, acc_p.at[pl.ds(s * ROWS_PER_SUB, ROWS_PER_SUB)])
    pltpu.sync_copy(z_hbm, acc_q.at[pl.ds(s * ROWS_PER_SUB, ROWS_PER_SUB)])
    plsc.subcore_barrier()

    def compute(b):
        R = rows_p.at[b]
        Qr = rows_q.at[b]

        def edge(e4, cc):
            for u in range(4):
                e = e4 * 4 + u
                for k in range(4):
                    mk = R[e, pl.ds(16 * k, 16)]
                    p = jnp.exp(mk * tvec - gvec[k])
                    R[e, pl.ds(16 * k, 16)] = p
                    Qr[e, pl.ds(16 * k, 16)] = mk * p
            return cc

        lax.fori_loop(0, CHUNK // 4, edge, 0)

    def stage(g, carry):
        pltpu.sync_copy(src_hbm.at[s, pl.ds(g * GSTAGE, GSTAGE)], sidx_v)
        pltpu.sync_copy(dst_hbm.at[s, pl.ds(g * GSTAGE, GSTAGE)], didx_v)

        for b in range(NSLOT):
            pltpu.async_copy(tbl.at[sidx_v.at[b]], rows_p.at[b], gsem.at[b])

        def grp(k, carry2):
            for b in range(NSLOT):
                j = k * NSLOT + b
                pltpu.make_async_copy(
                    tbl.at[sidx_v.at[j]], rows_p.at[b], gsem.at[b]).wait()
                compute(b)
                pltpu.async_copy(
                    rows_p.at[b], acc_p.at[didx_v.at[j]], psem.at[b],
                    add=True)
                pltpu.async_copy(
                    rows_q.at[b], acc_q.at[didx_v.at[j]], qsem.at[b],
                    add=True)

            @pl.when(k < GRP - 1)
            def _():
                for b in range(NSLOT):
                    j = (k + 1) * NSLOT + b
                    pltpu.make_async_copy(
                        rows_p.at[b], acc_p.at[didx_v.at[j - NSLOT]],
                        psem.at[b]).wait()
                    pltpu.make_async_copy(
                        rows_q.at[b], acc_q.at[didx_v.at[j - NSLOT]],
                        qsem.at[b]).wait()
                    pltpu.async_copy(
                        tbl.at[sidx_v.at[j]], rows_p.at[b], gsem.at[b])
            return carry2

        lax.fori_loop(0, GRP, grp, 0)
        for b in range(NSLOT):
            j = (GRP - 1) * NSLOT + b
            pltpu.make_async_copy(
                rows_p.at[b], acc_p.at[didx_v.at[j]], psem.at[b]).wait()
            pltpu.make_async_copy(
                rows_q.at[b], acc_q.at[didx_v.at[j]], qsem.at[b]).wait()
        return carry

    lax.fori_loop(0, NSTAGE, stage, 0)
    plsc.subcore_barrier()
    pltpu.sync_copy(
        acc_p.at[pl.ds(s * ROWS_PER_SUB, ROWS_PER_SUB)],
        op_hbm.at[pl.ds(c * ACC_ROWS + s * ROWS_PER_SUB, ROWS_PER_SUB)],
    )
    pltpu.sync_copy(
        acc_q.at[pl.ds(s * ROWS_PER_SUB, ROWS_PER_SUB)],
        oq_hbm.at[pl.ds(c * ACC_ROWS + s * ROWS_PER_SUB, ROWS_PER_SUB)],
    )


def kernel(x, edge_index, W_enc, b_enc, ln_g, ln_b, t, W1, b1, mg, mb, W2,
           b2, W_out, b_out):
    src = edge_index[0]
    dst = edge_index[1]
    src3 = src.reshape(NSUB, NCHUNK, CHUNK)
    dst3 = dst.reshape(NSUB, NCHUNK, CHUNK)
    zrows = jnp.zeros((ROWS_PER_SUB, 64), jnp.float32)

    xp = jnp.pad(x, ((0, 0), (0, DPAD - DIN)))
    wp = jnp.pad(W_enc, ((0, DPAD - DIN), (0, 0)))
    h = _encoder(xp, wp, b_enc.reshape(1, H))

    L = t.shape[0]
    for i in range(L):
        tb = jnp.full((1, H), t[i], jnp.float32)
        z, gb = _prep1(h, ln_g[i].reshape(1, H), ln_b[i].reshape(1, H), tb,
                       first=(i == 0))
        M, AUX = _prep2(z, gb, tb)
        Op, Oq = _edge_kernel(src3, dst3, M, AUX, zrows)
        Op = Op.reshape(2, ACC_ROWS, 64)
        Oq = Oq.reshape(2, ACC_ROWS, 64)
        h = _post(Op, Oq, z, h, W1[i], b1[i].reshape(1, 2 * H),
                  mg[i].reshape(1, 2 * H), mb[i].reshape(1, 2 * H), W2[i],
                  b2[i].reshape(1, H), residual=(i > 0))

    return _final(h, ln_g[0].reshape(1, H), ln_b[0].reshape(1, H),
                  W_out.reshape(1, H), b_out.reshape(1, 1))
